# baseline (speedup 1.0000x reference)
import functools

import jax
import jax.numpy as jnp
from jax import lax
from jax.experimental import pallas as pl
from jax.experimental.pallas import tpu as pltpu
from jax.experimental.pallas import tpu_sc as plsc

_NUM_CORES = 2
_NUM_SUBCORES = 16
_NW = _NUM_CORES * _NUM_SUBCORES
_CHUNK = 128
_NSLOT = 4


@functools.partial(jax.jit, static_argnums=(2, 3))
def _gather_flat(flat_idx, table, n, d):
    v = table.shape[0]
    per_w = n // _NW
    n_chunks = per_w // _CHUNK
    n_groups = n_chunks // _NSLOT
    stage_w = 5
    stage_rows = v // stage_w
    assert stage_rows % 8 == 0 and stage_rows * stage_w == v
    mesh = plsc.VectorSubcoreMesh(core_axis_name="c", subcore_axis_name="s")

    @functools.partial(
        pl.kernel,
        mesh=mesh,
        out_type=jax.ShapeDtypeStruct((n, d), jnp.float32),
        scratch_types=(
            [pltpu.VMEM((per_w,), jnp.int32),
             pltpu.VMEM((_NSLOT, _CHUNK, d), jnp.float32),
             pltpu.VMEM_SHARED((v, d), jnp.float32)]
            + [pltpu.SemaphoreType.DMA] * (2 * _NSLOT)
        ),
    )
    def emb(idx_hbm, table_hbm, out_hbm, idx_v, rows_v, table_sh, *sems):
        gsem = sems[:_NSLOT]
        osem = sems[_NSLOT:]
        sid = lax.axis_index("s")
        wid = sid * _NUM_CORES + lax.axis_index("c")
        base = wid * per_w

        @pl.when(sid < stage_w)
        def _():
            pltpu.sync_copy(
                table_hbm.at[pl.ds(sid * stage_rows, stage_rows)],
                table_sh.at[pl.ds(sid * stage_rows, stage_rows)],
            )

        pltpu.sync_copy(idx_hbm.at[pl.ds(base, per_w)], idx_v)
        plsc.subcore_barrier()

        def fire_gather(c, b):
            pltpu.async_copy(
                table_sh.at[idx_v.at[pl.ds(c * _CHUNK, _CHUNK)]],
                rows_v.at[b],
                gsem[b],
            )

        def wait_gather(b):
            pltpu.make_async_copy(
                table_sh.at[pl.ds(0, _CHUNK)], rows_v.at[b], gsem[b]
            ).wait()

        def fire_out(c, b):
            pltpu.async_copy(
                rows_v.at[b], out_hbm.at[pl.ds(base + c * _CHUNK, _CHUNK)], osem[b]
            )

        def wait_out(b):
            pltpu.make_async_copy(
                rows_v.at[b], out_hbm.at[pl.ds(base, _CHUNK)], osem[b]
            ).wait()

        for b in range(_NSLOT):
            fire_gather(b, b)

        def body(g, carry):
            c0 = g * _NSLOT
            for b in range(_NSLOT):
                wait_gather(b)
                fire_out(c0 + b, b)
            for b in range(_NSLOT):
                wait_out(b)
                fire_gather(c0 + _NSLOT + b, b)
            return carry

        lax.fori_loop(0, n_groups - 1, body, 0)

        c0 = (n_groups - 1) * _NSLOT
        for b in range(_NSLOT):
            wait_gather(b)
            fire_out(c0 + b, b)
        for b in range(_NSLOT):
            wait_out(b)

    return emb(flat_idx, table)


def kernel(t_index, pos_emb):
    b, t = t_index.shape
    d = pos_emb.shape[1]
    n = b * t
    flat = t_index.reshape(n)
    out = _gather_flat(flat, pos_emb, n, d)
    return out.reshape(b, t, d)
